# Initial kernel scaffold; baseline (speedup 1.0000x reference)
import jax
import jax.numpy as jnp
from jax.experimental import pallas as pl


def kernel(x, edge_index, edge_attr, W_src, b_src, W_dst, b_dst, W_edge, b_edge, W_attn, b_attn, gamma, beta, alpha):
    raise NotImplementedError("write your pallas kernel here")


# speedup vs baseline: 9.5271x; 9.5271x over previous
import functools

import jax
import jax.numpy as jnp
from jax import lax
from jax.experimental import pallas as pl
from jax.experimental.pallas import tpu as pltpu
from jax.experimental.pallas import tpu_sc as plsc

N = 10000
E = 320000
D = 128
NC = 2
NS = 16
NW = NC * NS
EPT = E // NW
C = 80
NCHUNK = EPT // C
G16 = EPT // 16
STRIPE = 640
LAST = N - 15 * STRIPE


def _node_proj_body(x_ref, ws_ref, bs_ref, w1_ref, wd_ref, w2_ref, bd_ref,
                    hx_ref, as_ref, ad_ref):
    x = x_ref[...]
    hx = jnp.dot(x, ws_ref[...], preferred_element_type=jnp.float32) + bs_ref[...]
    hx_ref[...] = hx
    as_ref[...] = jnp.dot(hx, w1_ref[...], preferred_element_type=jnp.float32)
    wd = jnp.dot(wd_ref[...], w2_ref[...], preferred_element_type=jnp.float32)
    cd = jnp.dot(bd_ref[...], w2_ref[...], preferred_element_type=jnp.float32)
    ad_ref[...] = jnp.dot(x, wd, preferred_element_type=jnp.float32) + cd


def _edge_proj_body(ea_ref, we_ref, w3_ref, be_ref, ba_ref, ae_ref):
    we = jnp.dot(we_ref[...], w3_ref[...], preferred_element_type=jnp.float32)
    ce = jnp.dot(be_ref[...], w3_ref[...], preferred_element_type=jnp.float32)
    ae_ref[...] = (jnp.dot(ea_ref[...], we, preferred_element_type=jnp.float32)
                   + ce + ba_ref[...])


def _finalize_body(acc_ref, den_ref, g_ref, b_ref, al_ref, out_ref):
    h = acc_ref[0] + acc_ref[1]
    d = den_ref[0] + den_ref[1]
    h = h / (d + 1e-16)
    mean = jnp.mean(h, axis=0, keepdims=True)
    var = jnp.mean((h - mean) * (h - mean), axis=0, keepdims=True)
    hbn = (h - mean) / jnp.sqrt(var + 1e-5) * g_ref[...] + b_ref[...]
    out_ref[...] = jnp.where(hbn > 0, hbn, al_ref[...] * hbn)


_sc_mesh = plsc.VectorSubcoreMesh(core_axis_name="c", subcore_axis_name="s")


@functools.partial(
    pl.kernel,
    mesh=_sc_mesh,
    compiler_params=pltpu.CompilerParams(needs_layout_passes=False),
    out_type=[
        jax.ShapeDtypeStruct((NC, N, D), jnp.float32),
        jax.ShapeDtypeStruct((NC * N,), jnp.float32),
    ],
    scratch_types=[
        pltpu.VMEM((C, D), jnp.float32),
        pltpu.VMEM((C,), jnp.int32),
        pltpu.VMEM((C,), jnp.int32),
        pltpu.VMEM((C,), jnp.float32),
        pltpu.VMEM((C,), jnp.float32),
        pltpu.VMEM((C,), jnp.float32),
        pltpu.VMEM((C,), jnp.float32),
        pltpu.VMEM((STRIPE,), jnp.float32),
        pltpu.VMEM_SHARED((N,), jnp.float32),
        pltpu.VMEM_SHARED((N,), jnp.float32),
        pltpu.VMEM_SHARED((N, D), jnp.float32),
        pltpu.VMEM_SHARED((N,), jnp.float32),
    ],
)
def _sc_aggregate(src_hbm, dst_hbm, ae_hbm, as_hbm, ad_hbm, hx_hbm,
                  acc_out, den_out,
                  rows, src_c, dst_c, ae_c, as_c, ad_c, ex_c, den_w,
                  as_sh, ad_sh, acc_sh, den_sh):
    c = lax.axis_index("c")
    s = lax.axis_index("s")
    w = s * NC + c
    base = w * EPT

    zero16 = jnp.zeros((16,), jnp.float32)

    def _zrow(i, _):
        for j in range(D // 16):
            rows[i, pl.ds(j * 16, 16)] = zero16
        return 0

    lax.fori_loop(0, C, _zrow, 0)
    for j in range(C // 16):
        ex_c[pl.ds(j * 16, 16)] = zero16

    def _stage_stripe(length):
        pltpu.sync_copy(as_hbm.at[pl.ds(s * STRIPE, length)],
                        den_w.at[pl.ds(0, length)])
        pltpu.sync_copy(den_w.at[pl.ds(0, length)],
                        as_sh.at[pl.ds(s * STRIPE, length)])
        pltpu.sync_copy(ad_hbm.at[pl.ds(s * STRIPE, length)],
                        den_w.at[pl.ds(0, length)])
        pltpu.sync_copy(den_w.at[pl.ds(0, length)],
                        ad_sh.at[pl.ds(s * STRIPE, length)])
        for k in range(length // C):
            pltpu.sync_copy(rows, acc_sh.at[pl.ds(s * STRIPE + k * C, C)])
            pltpu.sync_copy(ex_c, den_sh.at[pl.ds(s * STRIPE + k * C, C)])

    @pl.when(s < NS - 1)
    def _():
        _stage_stripe(STRIPE)

    @pl.when(s == NS - 1)
    def _():
        _stage_stripe(LAST)

    plsc.subcore_barrier()

    def _chunk(k, _):
        off = base + k * C
        pltpu.sync_copy(src_hbm.at[pl.ds(off, C)], src_c)
        pltpu.sync_copy(dst_hbm.at[pl.ds(off, C)], dst_c)
        pltpu.sync_copy(ae_hbm.at[pl.ds(off, C)], ae_c)
        pltpu.sync_copy(as_sh.at[src_c], as_c)
        pltpu.sync_copy(ad_sh.at[dst_c], ad_c)
        pltpu.sync_copy(hx_hbm.at[src_c], rows)
        for j in range(C // 16):
            sl = pl.ds(j * 16, 16)
            ex_c[sl] = jnp.exp(as_c[sl] + ad_c[sl] + ae_c[sl])

        def _scale(i, _):
            ev = plsc.load_gather(ex_c, [lax.broadcast(i, (16,))])
            for j in range(D // 16):
                rows[i, pl.ds(j * 16, 16)] = rows[i, pl.ds(j * 16, 16)] * ev
            return 0

        lax.fori_loop(0, C, _scale, 0)
        pltpu.sync_copy(rows, acc_sh.at[dst_c], add=True)
        pltpu.sync_copy(ex_c, den_sh.at[dst_c], add=True)
        return 0

    lax.fori_loop(0, NCHUNK, _chunk, 0)

    plsc.subcore_barrier()

    @pl.when(s < NS - 1)
    def _():
        pltpu.sync_copy(acc_sh.at[pl.ds(s * STRIPE, STRIPE)],
                        acc_out.at[c, pl.ds(s * STRIPE, STRIPE)])
        pltpu.sync_copy(den_sh.at[pl.ds(s * STRIPE, STRIPE)], den_w)
        pltpu.sync_copy(den_w, den_out.at[pl.ds(c * N + s * STRIPE, STRIPE)])

    @pl.when(s == NS - 1)
    def _():
        pltpu.sync_copy(acc_sh.at[pl.ds(s * STRIPE, LAST)],
                        acc_out.at[c, pl.ds(s * STRIPE, LAST)])
        pltpu.sync_copy(den_sh.at[pl.ds(s * STRIPE, LAST)],
                        den_w.at[pl.ds(0, LAST)])
        pltpu.sync_copy(den_w.at[pl.ds(0, LAST)],
                        den_out.at[pl.ds(c * N + s * STRIPE, LAST)])


def kernel(x, edge_index, edge_attr, W_src, b_src, W_dst, b_dst,
           W_edge, b_edge, W_attn, b_attn, gamma, beta, alpha):
    src = edge_index[0]
    dst = edge_index[1]
    w1 = W_attn[0:D]
    w2 = W_attn[D:2 * D]
    w3 = W_attn[2 * D:3 * D]

    hx, as_n, ad_n = pl.pallas_call(
        _node_proj_body,
        out_shape=[
            jax.ShapeDtypeStruct((N, D), jnp.float32),
            jax.ShapeDtypeStruct((N, 1), jnp.float32),
            jax.ShapeDtypeStruct((N, 1), jnp.float32),
        ],
    )(x, W_src, b_src.reshape(1, D), w1, W_dst, w2, b_dst.reshape(1, D))

    BE = 6400
    ae = pl.pallas_call(
        _edge_proj_body,
        grid=(E // BE,),
        in_specs=[
            pl.BlockSpec((BE, 11), lambda i: (i, 0)),
            pl.BlockSpec((11, D), lambda i: (0, 0)),
            pl.BlockSpec((D, 1), lambda i: (0, 0)),
            pl.BlockSpec((1, D), lambda i: (0, 0)),
            pl.BlockSpec((1, 1), lambda i: (0, 0)),
        ],
        out_specs=pl.BlockSpec((BE, 1), lambda i: (i, 0)),
        out_shape=jax.ShapeDtypeStruct((E, 1), jnp.float32),
    )(edge_attr, W_edge, w3, b_edge.reshape(1, D), b_attn.reshape(1, 1))

    acc, den = _sc_aggregate(src, dst, ae.reshape(E), as_n.reshape(N),
                             ad_n.reshape(N), hx)

    out = pl.pallas_call(
        _finalize_body,
        out_shape=jax.ShapeDtypeStruct((N, D), jnp.float32),
    )(acc, den.reshape(NC, N, 1), gamma.reshape(1, D), beta.reshape(1, D),
      alpha.reshape(1, 1))
    return out


# speedup vs baseline: 13.2344x; 1.3891x over previous
import functools

import jax
import jax.numpy as jnp
from jax import lax
from jax.experimental import pallas as pl
from jax.experimental.pallas import tpu as pltpu
from jax.experimental.pallas import tpu_sc as plsc

N = 10000
E = 320000
D = 128
EA = 11
NC = 2
NS = 16
NW = NC * NS
EPT = E // NW
C = 80
C11 = C * EA
NCHUNK = EPT // C
STRIPE = 640
LAST = N - (NS - 1) * STRIPE


def _node_proj_body(x_ref, ws_ref, bs_ref, w1_ref, wd_ref, w2_ref, bd_ref,
                    we_ref, w3_ref, be_ref, ba_ref,
                    hx_ref, as_ref, ad_ref, wep_ref):
    x = x_ref[...]
    hx = jnp.dot(x, ws_ref[...], preferred_element_type=jnp.float32) + bs_ref[...]
    hx_ref[...] = hx
    ce = (jnp.dot(be_ref[...], w3_ref[...], preferred_element_type=jnp.float32)
          + ba_ref[...])
    as_ref[...] = jnp.dot(hx, w1_ref[...], preferred_element_type=jnp.float32) + ce
    wd = jnp.dot(wd_ref[...], w2_ref[...], preferred_element_type=jnp.float32)
    cd = jnp.dot(bd_ref[...], w2_ref[...], preferred_element_type=jnp.float32)
    ad_ref[...] = jnp.dot(x, wd, preferred_element_type=jnp.float32) + cd
    we = jnp.dot(we_ref[...], w3_ref[...], preferred_element_type=jnp.float32)
    wep_ref[...] = jnp.concatenate(
        [jnp.zeros((1, 1), jnp.float32), we,
         jnp.zeros((15 - EA, 1), jnp.float32)], axis=0)


def _finalize_body(acc_ref, den_ref, g_ref, b_ref, al_ref, out_ref):
    h = acc_ref[0] + acc_ref[1]
    d = den_ref[0] + den_ref[1]
    h = h / (d + 1e-16)
    mean = jnp.mean(h, axis=0, keepdims=True)
    var = jnp.mean((h - mean) * (h - mean), axis=0, keepdims=True)
    hbn = (h - mean) / jnp.sqrt(var + 1e-5) * g_ref[...] + b_ref[...]
    out_ref[...] = jnp.where(hbn > 0, hbn, al_ref[...] * hbn)


_sc_mesh = plsc.VectorSubcoreMesh(core_axis_name="c", subcore_axis_name="s")


@functools.partial(
    pl.kernel,
    mesh=_sc_mesh,
    compiler_params=pltpu.CompilerParams(needs_layout_passes=False),
    out_type=[
        jax.ShapeDtypeStruct((NC, N, D), jnp.float32),
        jax.ShapeDtypeStruct((NC * N,), jnp.float32),
    ],
    scratch_types=[
        pltpu.VMEM((N,), jnp.float32),
        pltpu.VMEM((N,), jnp.float32),
        pltpu.VMEM((C, D), jnp.float32),
        pltpu.VMEM((C, D), jnp.float32),
        pltpu.VMEM((C11,), jnp.float32),
        pltpu.VMEM((C11,), jnp.float32),
        pltpu.VMEM((C,), jnp.float32),
        pltpu.VMEM((C,), jnp.float32),
        pltpu.VMEM((C,), jnp.float32),
        pltpu.VMEM((C,), jnp.float32),
        pltpu.VMEM((C,), jnp.float32),
        pltpu.VMEM((C,), jnp.float32),
        pltpu.VMEM((C,), jnp.int32),
        pltpu.VMEM((C,), jnp.int32),
        pltpu.VMEM((C,), jnp.int32),
        pltpu.VMEM((C,), jnp.int32),
        pltpu.VMEM((16,), jnp.float32),
        pltpu.VMEM((STRIPE,), jnp.float32),
        pltpu.VMEM_SHARED((N, D), jnp.float32),
        pltpu.VMEM_SHARED((N,), jnp.float32),
        pltpu.SemaphoreType.DMA,
        pltpu.SemaphoreType.DMA,
        pltpu.SemaphoreType.DMA,
        pltpu.SemaphoreType.DMA,
        pltpu.SemaphoreType.DMA,
        pltpu.SemaphoreType.DMA,
        pltpu.SemaphoreType.DMA,
        pltpu.SemaphoreType.DMA,
    ],
)
def _sc_aggregate(src_hbm, dst_hbm, ea_hbm, as_hbm, ad_hbm, hx_hbm, we_hbm,
                  acc_out, den_out,
                  as_l, ad_l, rows0, rows1, ea0, ea1, as0, as1, ad0, ad1,
                  ex0, ex1, dstc0, dstc1, srcc0, srcc1, we_v, den_w,
                  acc_sh, den_sh,
                  sem_ea0, sem_ea1, sem_as0, sem_as1, sem_ad0, sem_ad1,
                  sem_rw0, sem_rw1):
    c = lax.axis_index("c")
    s = lax.axis_index("s")
    w = s * NC + c
    base = w * EPT

    rows = (rows0, rows1)
    eab = (ea0, ea1)
    asb = (as0, as1)
    adb = (ad0, ad1)
    exb = (ex0, ex1)
    dstc = (dstc0, dstc1)
    srcc = (srcc0, srcc1)
    sem_ea = (sem_ea0, sem_ea1)
    sem_as = (sem_as0, sem_as1)
    sem_ad = (sem_ad0, sem_ad1)
    sem_rw = (sem_rw0, sem_rw1)

    pltpu.sync_copy(as_hbm, as_l)
    pltpu.sync_copy(ad_hbm, ad_l)
    pltpu.sync_copy(we_hbm, we_v)
    wkv = [plsc.load_gather(we_v, [jnp.full((16,), k + 1, jnp.int32)])
           for k in range(EA)]
    ii11 = lax.iota(jnp.int32, 16) * EA

    zero16 = jnp.zeros((16,), jnp.float32)

    def _zrow(i, _):
        for j in range(D // 16):
            rows0[i, pl.ds(j * 16, 16)] = zero16
        return 0

    lax.fori_loop(0, C, _zrow, 0)
    for j in range(C // 16):
        ex0[pl.ds(j * 16, 16)] = zero16

    def _stage_stripe(length):
        for k in range(length // C):
            pltpu.sync_copy(rows0, acc_sh.at[pl.ds(s * STRIPE + k * C, C)])
            pltpu.sync_copy(ex0, den_sh.at[pl.ds(s * STRIPE + k * C, C)])

    @pl.when(s < NS - 1)
    def _():
        _stage_stripe(STRIPE)

    @pl.when(s == NS - 1)
    def _():
        _stage_stripe(LAST)

    plsc.subcore_barrier()

    def _issue(i, b):
        g = base + i * C
        pltpu.async_copy(src_hbm.at[pl.ds(g, C)], srcc[b], sem_as[b])
        pltpu.async_copy(dst_hbm.at[pl.ds(g, C)], dstc[b], sem_ad[b])
        pltpu.async_copy(ea_hbm.at[pl.ds(g * EA, C11)], eab[b], sem_ea[b])

    def _wait_idx(i, b):
        g = base + i * C
        pltpu.make_async_copy(src_hbm.at[pl.ds(g, C)], srcc[b],
                              sem_as[b]).wait()
        pltpu.make_async_copy(dst_hbm.at[pl.ds(g, C)], dstc[b],
                              sem_ad[b]).wait()

    def _issue_rows(i, b):
        pltpu.async_copy(hx_hbm.at[srcc[b]], rows[b], sem_rw[b])

    def _wait_in(i, b):
        g = base + i * C
        pltpu.make_async_copy(ea_hbm.at[pl.ds(g * EA, C11)], eab[b],
                              sem_ea[b]).wait()
        pltpu.make_async_copy(hx_hbm.at[srcc[b]], rows[b], sem_rw[b]).wait()

    def _compute(i, b):
        for j in range(C // 16):
            sl = pl.ds(j * 16, 16)
            ae16 = zero16
            for k in range(EA):
                idx = ii11 + (j * 16 * EA + k)
                ae16 = ae16 + wkv[k] * plsc.load_gather(eab[b], [idx])
            av = plsc.load_gather(as_l, [srcc[b][sl]])
            dv = plsc.load_gather(ad_l, [dstc[b][sl]])
            exb[b][sl] = jnp.exp(av + dv + ae16)

        def _scale(i2, _):
            ev = plsc.load_gather(exb[b], [lax.broadcast(i2, (16,))])
            for jj in range(D // 16):
                sl2 = pl.ds(jj * 16, 16)
                rows[b][i2, sl2] = rows[b][i2, sl2] * ev
            return 0

        lax.fori_loop(0, C, _scale, 0, unroll=2)
        pltpu.sync_copy(rows[b], acc_sh.at[dstc[b]], add=True)
        pltpu.sync_copy(exb[b], den_sh.at[dstc[b]], add=True)

    def _body(i, _):
        _issue(i, 0)
        _wait_idx(i, 0)
        _issue_rows(i, 0)
        _wait_in(i, 0)
        _compute(i, 0)
        return 0

    lax.fori_loop(0, NCHUNK, _body, 0)

    plsc.subcore_barrier()

    @pl.when(s < NS - 1)
    def _():
        pltpu.sync_copy(acc_sh.at[pl.ds(s * STRIPE, STRIPE)],
                        acc_out.at[c, pl.ds(s * STRIPE, STRIPE)])
        pltpu.sync_copy(den_sh.at[pl.ds(s * STRIPE, STRIPE)], den_w)
        pltpu.sync_copy(den_w, den_out.at[pl.ds(c * N + s * STRIPE, STRIPE)])

    @pl.when(s == NS - 1)
    def _():
        pltpu.sync_copy(acc_sh.at[pl.ds(s * STRIPE, LAST)],
                        acc_out.at[c, pl.ds(s * STRIPE, LAST)])
        pltpu.sync_copy(den_sh.at[pl.ds(s * STRIPE, LAST)],
                        den_w.at[pl.ds(0, LAST)])
        pltpu.sync_copy(den_w.at[pl.ds(0, LAST)],
                        den_out.at[pl.ds(c * N + s * STRIPE, LAST)])


def kernel(x, edge_index, edge_attr, W_src, b_src, W_dst, b_dst,
           W_edge, b_edge, W_attn, b_attn, gamma, beta, alpha):
    src = edge_index[0]
    dst = edge_index[1]
    ea_flat = edge_attr.reshape(E * EA)
    w1 = W_attn[0:D]
    w2 = W_attn[D:2 * D]
    w3 = W_attn[2 * D:3 * D]

    hx, as_n, ad_n, we16 = pl.pallas_call(
        _node_proj_body,
        out_shape=[
            jax.ShapeDtypeStruct((N, D), jnp.float32),
            jax.ShapeDtypeStruct((N, 1), jnp.float32),
            jax.ShapeDtypeStruct((N, 1), jnp.float32),
            jax.ShapeDtypeStruct((16, 1), jnp.float32),
        ],
    )(x, W_src, b_src.reshape(1, D), w1, W_dst, w2, b_dst.reshape(1, D),
      W_edge, w3, b_edge.reshape(1, D), b_attn.reshape(1, 1))

    acc, den = _sc_aggregate(src, dst, ea_flat, as_n.reshape(N),
                             ad_n.reshape(N), hx, we16.reshape(16))

    out = pl.pallas_call(
        _finalize_body,
        out_shape=jax.ShapeDtypeStruct((N, D), jnp.float32),
    )(acc, den.reshape(NC, N, 1), gamma.reshape(1, D), beta.reshape(1, D),
      alpha.reshape(1, 1))
    return out


# speedup vs baseline: 16.8803x; 1.2755x over previous
import functools

import jax
import jax.numpy as jnp
from jax import lax
from jax.experimental import pallas as pl
from jax.experimental.pallas import tpu as pltpu
from jax.experimental.pallas import tpu_sc as plsc

N = 10000
E = 320000
D = 128
EA = 11
NC = 2
NS = 16
NW = NC * NS
EPT = E // NW
C = 80
C11 = C * EA
NCHUNK = EPT // C
STRIPE = 640
LAST = N - (NS - 1) * STRIPE


def _node_proj_body(x_ref, ws_ref, bs_ref, w1_ref, wd_ref, w2_ref, bd_ref,
                    we_ref, w3_ref, be_ref, ba_ref,
                    hx_ref, as_ref, ad_ref, wep_ref):
    x = x_ref[...]
    hx = jnp.dot(x, ws_ref[...], preferred_element_type=jnp.float32) + bs_ref[...]
    hx_ref[...] = hx
    ce = (jnp.dot(be_ref[...], w3_ref[...], preferred_element_type=jnp.float32)
          + ba_ref[...])
    as_ref[...] = jnp.dot(hx, w1_ref[...], preferred_element_type=jnp.float32) + ce
    wd = jnp.dot(wd_ref[...], w2_ref[...], preferred_element_type=jnp.float32)
    cd = jnp.dot(bd_ref[...], w2_ref[...], preferred_element_type=jnp.float32)
    ad_ref[...] = jnp.dot(x, wd, preferred_element_type=jnp.float32) + cd
    we = jnp.dot(we_ref[...], w3_ref[...], preferred_element_type=jnp.float32)
    wep_ref[...] = jnp.concatenate(
        [jnp.zeros((1, 1), jnp.float32), we,
         jnp.zeros((15 - EA, 1), jnp.float32)], axis=0)


def _finalize_body(acc_ref, den_ref, g_ref, b_ref, al_ref, out_ref):
    h = acc_ref[0] + acc_ref[1]
    d = den_ref[0] + den_ref[1]
    h = h / (d + 1e-16)
    mean = jnp.mean(h, axis=0, keepdims=True)
    var = jnp.mean((h - mean) * (h - mean), axis=0, keepdims=True)
    hbn = (h - mean) / jnp.sqrt(var + 1e-5) * g_ref[...] + b_ref[...]
    out_ref[...] = jnp.where(hbn > 0, hbn, al_ref[...] * hbn)


_sc_mesh = plsc.VectorSubcoreMesh(core_axis_name="c", subcore_axis_name="s")


@functools.partial(
    pl.kernel,
    mesh=_sc_mesh,
    compiler_params=pltpu.CompilerParams(needs_layout_passes=False),
    out_type=[
        jax.ShapeDtypeStruct((NC, N, D), jnp.float32),
        jax.ShapeDtypeStruct((NC * N,), jnp.float32),
    ],
    scratch_types=[
        pltpu.VMEM((N,), jnp.float32),
        pltpu.VMEM((N,), jnp.float32),
        pltpu.VMEM((C, D), jnp.float32),
        pltpu.VMEM((C, D), jnp.float32),
        pltpu.VMEM((C11,), jnp.float32),
        pltpu.VMEM((C11,), jnp.float32),
        pltpu.VMEM((C,), jnp.float32),
        pltpu.VMEM((C,), jnp.float32),
        pltpu.VMEM((C,), jnp.int32),
        pltpu.VMEM((C,), jnp.int32),
        pltpu.VMEM((C,), jnp.int32),
        pltpu.VMEM((C,), jnp.int32),
        pltpu.VMEM((16,), jnp.float32),
        pltpu.VMEM((STRIPE,), jnp.float32),
        pltpu.VMEM_SHARED((N, D), jnp.float32),
        pltpu.VMEM_SHARED((N,), jnp.float32),
        pltpu.SemaphoreType.DMA,
        pltpu.SemaphoreType.DMA,
        pltpu.SemaphoreType.DMA,
        pltpu.SemaphoreType.DMA,
        pltpu.SemaphoreType.DMA,
        pltpu.SemaphoreType.DMA,
        pltpu.SemaphoreType.DMA,
        pltpu.SemaphoreType.DMA,
    ],
)
def _sc_aggregate(src_hbm, dst_hbm, ea_hbm, as_hbm, ad_hbm, hx_hbm, we_hbm,
                  acc_out, den_out,
                  as_l, ad_l, rows0, rows1, ea0, ea1,
                  ex0, ex1, dstc0, dstc1, srcc0, srcc1, we_v, den_w,
                  acc_sh, den_sh,
                  sem_ea0, sem_ea1, sem_as0, sem_as1, sem_ad0, sem_ad1,
                  sem_rw0, sem_rw1):
    c = lax.axis_index("c")
    s = lax.axis_index("s")
    w = s * NC + c
    base = w * EPT

    rows = (rows0, rows1)
    eab = (ea0, ea1)
    exb = (ex0, ex1)
    dstc = (dstc0, dstc1)
    srcc = (srcc0, srcc1)
    sem_ea = (sem_ea0, sem_ea1)
    sem_as = (sem_as0, sem_as1)
    sem_ad = (sem_ad0, sem_ad1)
    sem_rw = (sem_rw0, sem_rw1)

    pltpu.sync_copy(as_hbm, as_l)
    pltpu.sync_copy(ad_hbm, ad_l)
    pltpu.sync_copy(we_hbm, we_v)
    wkv = [plsc.load_gather(we_v, [jnp.full((16,), k + 1, jnp.int32)])
           for k in range(EA)]
    ii11 = lax.iota(jnp.int32, 16) * EA

    zero16 = jnp.zeros((16,), jnp.float32)

    def _zrow(i, _):
        for j in range(D // 16):
            rows0[i, pl.ds(j * 16, 16)] = zero16
        return 0

    lax.fori_loop(0, C, _zrow, 0)
    for j in range(C // 16):
        ex0[pl.ds(j * 16, 16)] = zero16

    def _stage_stripe(length):
        for k in range(length // C):
            pltpu.sync_copy(rows0, acc_sh.at[pl.ds(s * STRIPE + k * C, C)])
            pltpu.sync_copy(ex0, den_sh.at[pl.ds(s * STRIPE + k * C, C)])

    @pl.when(s < NS - 1)
    def _():
        _stage_stripe(STRIPE)

    @pl.when(s == NS - 1)
    def _():
        _stage_stripe(LAST)

    plsc.subcore_barrier()

    def _issue(i, b):
        g = base + i * C
        pltpu.async_copy(src_hbm.at[pl.ds(g, C)], srcc[b], sem_as[b])
        pltpu.async_copy(dst_hbm.at[pl.ds(g, C)], dstc[b], sem_ad[b])
        pltpu.async_copy(ea_hbm.at[pl.ds(g * EA, C11)], eab[b], sem_ea[b])

    def _wait_idx(i, b):
        g = base + i * C
        pltpu.make_async_copy(src_hbm.at[pl.ds(g, C)], srcc[b],
                              sem_as[b]).wait()
        pltpu.make_async_copy(dst_hbm.at[pl.ds(g, C)], dstc[b],
                              sem_ad[b]).wait()

    def _issue_rows(i, b):
        pltpu.async_copy(hx_hbm.at[srcc[b]], rows[b], sem_rw[b])

    def _wait_in(i, b):
        g = base + i * C
        pltpu.make_async_copy(ea_hbm.at[pl.ds(g * EA, C11)], eab[b],
                              sem_ea[b]).wait()
        pltpu.make_async_copy(hx_hbm.at[srcc[b]], rows[b], sem_rw[b]).wait()

    def _compute(i, b):
        for j in range(C // 16):
            sl = pl.ds(j * 16, 16)
            ae16 = zero16
            for k in range(EA):
                idx = ii11 + (j * 16 * EA + k)
                ae16 = ae16 + wkv[k] * plsc.load_gather(eab[b], [idx])
            av = plsc.load_gather(as_l, [srcc[b][sl]])
            dv = plsc.load_gather(ad_l, [dstc[b][sl]])
            exb[b][sl] = jnp.exp(av + dv + ae16)

        def _scale(i2, _):
            ev = plsc.load_gather(exb[b], [lax.broadcast(i2, (16,))])
            for jj in range(D // 16):
                sl2 = pl.ds(jj * 16, 16)
                rows[b][i2, sl2] = rows[b][i2, sl2] * ev
            return 0

        lax.fori_loop(0, C, _scale, 0, unroll=2)
        pltpu.sync_copy(rows[b], acc_sh.at[dstc[b]], add=True)
        pltpu.sync_copy(exb[b], den_sh.at[dstc[b]], add=True)

    _issue(0, 0)
    _issue(1, 1)
    _wait_idx(0, 0)
    _issue_rows(0, 0)

    def _body(k2, _):
        i0 = 2 * k2
        i1 = i0 + 1
        _wait_idx(i1, 1)
        _issue_rows(i1, 1)
        _wait_in(i0, 0)
        _compute(i0, 0)
        _issue(i0 + 2, 0)
        _wait_idx(i0 + 2, 0)
        _issue_rows(i0 + 2, 0)
        _wait_in(i1, 1)
        _compute(i1, 1)
        _issue(i1 + 2, 1)
        return 0

    lax.fori_loop(0, (NCHUNK - 3) // 2, _body, 0)

    _wait_idx(NCHUNK - 2, 1)
    _issue_rows(NCHUNK - 2, 1)
    _wait_in(NCHUNK - 3, 0)
    _compute(NCHUNK - 3, 0)
    _issue(NCHUNK - 1, 0)
    _wait_idx(NCHUNK - 1, 0)
    _issue_rows(NCHUNK - 1, 0)
    _wait_in(NCHUNK - 2, 1)
    _compute(NCHUNK - 2, 1)
    _wait_in(NCHUNK - 1, 0)
    _compute(NCHUNK - 1, 0)

    plsc.subcore_barrier()

    @pl.when(s < NS - 1)
    def _():
        pltpu.sync_copy(acc_sh.at[pl.ds(s * STRIPE, STRIPE)],
                        acc_out.at[c, pl.ds(s * STRIPE, STRIPE)])
        pltpu.sync_copy(den_sh.at[pl.ds(s * STRIPE, STRIPE)], den_w)
        pltpu.sync_copy(den_w, den_out.at[pl.ds(c * N + s * STRIPE, STRIPE)])

    @pl.when(s == NS - 1)
    def _():
        pltpu.sync_copy(acc_sh.at[pl.ds(s * STRIPE, LAST)],
                        acc_out.at[c, pl.ds(s * STRIPE, LAST)])
        pltpu.sync_copy(den_sh.at[pl.ds(s * STRIPE, LAST)],
                        den_w.at[pl.ds(0, LAST)])
        pltpu.sync_copy(den_w.at[pl.ds(0, LAST)],
                        den_out.at[pl.ds(c * N + s * STRIPE, LAST)])


def kernel(x, edge_index, edge_attr, W_src, b_src, W_dst, b_dst,
           W_edge, b_edge, W_attn, b_attn, gamma, beta, alpha):
    src = edge_index[0]
    dst = edge_index[1]
    ea_flat = edge_attr.reshape(E * EA)
    w1 = W_attn[0:D]
    w2 = W_attn[D:2 * D]
    w3 = W_attn[2 * D:3 * D]

    hx, as_n, ad_n, we16 = pl.pallas_call(
        _node_proj_body,
        out_shape=[
            jax.ShapeDtypeStruct((N, D), jnp.float32),
            jax.ShapeDtypeStruct((N, 1), jnp.float32),
            jax.ShapeDtypeStruct((N, 1), jnp.float32),
            jax.ShapeDtypeStruct((16, 1), jnp.float32),
        ],
    )(x, W_src, b_src.reshape(1, D), w1, W_dst, w2, b_dst.reshape(1, D),
      W_edge, w3, b_edge.reshape(1, D), b_attn.reshape(1, 1))

    acc, den = _sc_aggregate(src, dst, ea_flat, as_n.reshape(N),
                             ad_n.reshape(N), hx, we16.reshape(16))

    out = pl.pallas_call(
        _finalize_body,
        out_shape=jax.ShapeDtypeStruct((N, D), jnp.float32),
    )(acc, den.reshape(NC, N, 1), gamma.reshape(1, D), beta.reshape(1, D),
      alpha.reshape(1, 1))
    return out


# speedup vs baseline: 16.9274x; 1.0028x over previous
import functools

import jax
import jax.numpy as jnp
from jax import lax
from jax.experimental import pallas as pl
from jax.experimental.pallas import tpu as pltpu
from jax.experimental.pallas import tpu_sc as plsc

N = 10000
E = 320000
D = 128
EA = 11
NC = 2
NS = 16
NW = NC * NS
EPT = E // NW
C = 80
C11 = C * EA
NCHUNK = EPT // C
STRIPE = 640
LAST = N - (NS - 1) * STRIPE


def _node_proj_body(x_ref, ws_ref, bs_ref, w1_ref, wd_ref, w2_ref, bd_ref,
                    we_ref, w3_ref, be_ref, ba_ref,
                    hx_ref, as_ref, ad_ref, wep_ref):
    x = x_ref[...]
    hx = jnp.dot(x, ws_ref[...], preferred_element_type=jnp.float32) + bs_ref[...]
    hx_ref[...] = hx
    ce = (jnp.dot(be_ref[...], w3_ref[...], preferred_element_type=jnp.float32)
          + ba_ref[...])
    as_ref[...] = jnp.dot(hx, w1_ref[...], preferred_element_type=jnp.float32) + ce
    wd = jnp.dot(wd_ref[...], w2_ref[...], preferred_element_type=jnp.float32)
    cd = jnp.dot(bd_ref[...], w2_ref[...], preferred_element_type=jnp.float32)
    ad_ref[...] = jnp.dot(x, wd, preferred_element_type=jnp.float32) + cd
    we = jnp.dot(we_ref[...], w3_ref[...], preferred_element_type=jnp.float32)
    wep_ref[...] = jnp.concatenate(
        [jnp.zeros((1, 1), jnp.float32), we,
         jnp.zeros((15 - EA, 1), jnp.float32)], axis=0)


def _finalize_body(acc_ref, den_ref, g_ref, b_ref, al_ref, out_ref):
    h = acc_ref[0] + acc_ref[1]
    d = den_ref[0] + den_ref[1]
    h = h / (d + 1e-16)
    mean = jnp.mean(h, axis=0, keepdims=True)
    var = jnp.mean((h - mean) * (h - mean), axis=0, keepdims=True)
    hbn = (h - mean) / jnp.sqrt(var + 1e-5) * g_ref[...] + b_ref[...]
    out_ref[...] = jnp.where(hbn > 0, hbn, al_ref[...] * hbn)


_sc_mesh = plsc.VectorSubcoreMesh(core_axis_name="c", subcore_axis_name="s")


@functools.partial(
    pl.kernel,
    mesh=_sc_mesh,
    compiler_params=pltpu.CompilerParams(needs_layout_passes=False),
    out_type=[
        jax.ShapeDtypeStruct((NC, N, D), jnp.float32),
        jax.ShapeDtypeStruct((NC * N,), jnp.float32),
    ],
    scratch_types=[
        pltpu.VMEM((N,), jnp.float32),
        pltpu.VMEM((N,), jnp.float32),
        pltpu.VMEM((C, D), jnp.float32),
        pltpu.VMEM((C, D), jnp.float32),
        pltpu.VMEM((C11,), jnp.float32),
        pltpu.VMEM((C11,), jnp.float32),
        pltpu.VMEM((C,), jnp.float32),
        pltpu.VMEM((C,), jnp.float32),
        pltpu.VMEM((C,), jnp.int32),
        pltpu.VMEM((C,), jnp.int32),
        pltpu.VMEM((C,), jnp.int32),
        pltpu.VMEM((C,), jnp.int32),
        pltpu.VMEM((16,), jnp.float32),
        pltpu.VMEM((STRIPE,), jnp.float32),
        pltpu.VMEM_SHARED((N, D), jnp.float32),
        pltpu.VMEM_SHARED((N,), jnp.float32),
        pltpu.SemaphoreType.DMA,
        pltpu.SemaphoreType.DMA,
        pltpu.SemaphoreType.DMA,
        pltpu.SemaphoreType.DMA,
        pltpu.SemaphoreType.DMA,
        pltpu.SemaphoreType.DMA,
        pltpu.SemaphoreType.DMA,
        pltpu.SemaphoreType.DMA,
    ],
)
def _sc_aggregate(src_hbm, dst_hbm, ea_hbm, as_hbm, ad_hbm, hx_hbm, we_hbm,
                  acc_out, den_out,
                  as_l, ad_l, rows0, rows1, ea0, ea1,
                  ex0, ex1, dstc0, dstc1, srcc0, srcc1, we_v, den_w,
                  acc_sh, den_sh,
                  sem_ea0, sem_ea1, sem_as0, sem_as1, sem_ad0, sem_ad1,
                  sem_rw0, sem_rw1):
    c = lax.axis_index("c")
    s = lax.axis_index("s")
    w = s * NC + c
    base = w * EPT

    rows = (rows0, rows1)
    eab = (ea0, ea1)
    exb = (ex0, ex1)
    dstc = (dstc0, dstc1)
    srcc = (srcc0, srcc1)
    sem_ea = (sem_ea0, sem_ea1)
    sem_as = (sem_as0, sem_as1)
    sem_ad = (sem_ad0, sem_ad1)
    sem_rw = (sem_rw0, sem_rw1)

    pltpu.sync_copy(as_hbm, as_l)
    pltpu.sync_copy(ad_hbm, ad_l)
    pltpu.sync_copy(we_hbm, we_v)
    wkv = [plsc.load_gather(we_v, [jnp.full((16,), k + 1, jnp.int32)])
           for k in range(EA)]
    ii11 = lax.iota(jnp.int32, 16) * EA

    zero16 = jnp.zeros((16,), jnp.float32)

    def _zrow(i, _):
        for j in range(D // 16):
            rows0[i, pl.ds(j * 16, 16)] = zero16
        return 0

    lax.fori_loop(0, C, _zrow, 0)
    for j in range(C // 16):
        ex0[pl.ds(j * 16, 16)] = zero16

    def _stage_stripe(length):
        for k in range(length // C):
            pltpu.sync_copy(rows0, acc_sh.at[pl.ds(s * STRIPE + k * C, C)])
            pltpu.sync_copy(ex0, den_sh.at[pl.ds(s * STRIPE + k * C, C)])

    @pl.when(s < NS - 1)
    def _():
        _stage_stripe(STRIPE)

    @pl.when(s == NS - 1)
    def _():
        _stage_stripe(LAST)

    plsc.subcore_barrier()

    def _issue(i, b):
        g = base + i * C
        pltpu.async_copy(src_hbm.at[pl.ds(g, C)], srcc[b], sem_as[b])
        pltpu.async_copy(dst_hbm.at[pl.ds(g, C)], dstc[b], sem_ad[b])
        pltpu.async_copy(ea_hbm.at[pl.ds(g * EA, C11)], eab[b], sem_ea[b])

    def _wait_idx(i, b):
        g = base + i * C
        pltpu.make_async_copy(src_hbm.at[pl.ds(g, C)], srcc[b],
                              sem_as[b]).wait()
        pltpu.make_async_copy(dst_hbm.at[pl.ds(g, C)], dstc[b],
                              sem_ad[b]).wait()

    def _issue_rows(i, b):
        pltpu.async_copy(hx_hbm.at[srcc[b]], rows[b], sem_rw[b])

    def _wait_in(i, b):
        g = base + i * C
        pltpu.make_async_copy(ea_hbm.at[pl.ds(g * EA, C11)], eab[b],
                              sem_ea[b]).wait()
        pltpu.make_async_copy(hx_hbm.at[srcc[b]], rows[b], sem_rw[b]).wait()

    def _compute(i, b):
        for j in range(C // 16):
            sl = pl.ds(j * 16, 16)
            ae16 = zero16
            for k in range(EA):
                idx = ii11 + (j * 16 * EA + k)
                ae16 = ae16 + wkv[k] * plsc.load_gather(eab[b], [idx])
            av = plsc.load_gather(as_l, [srcc[b][sl]])
            dv = plsc.load_gather(ad_l, [dstc[b][sl]])
            exb[b][sl] = jnp.exp(av + dv + ae16)

        def _scale(i2, _):
            ev = plsc.load_gather(exb[b], [lax.broadcast(i2, (16,))])
            for jj in range(D // 16):
                sl2 = pl.ds(jj * 16, 16)
                rows[b][i2, sl2] = rows[b][i2, sl2] * ev
            return 0

        lax.fori_loop(0, C, _scale, 0, unroll=4)
        pltpu.sync_copy(rows[b], acc_sh.at[dstc[b]], add=True)
        pltpu.sync_copy(exb[b], den_sh.at[dstc[b]], add=True)

    _issue(0, 0)
    _issue(1, 1)
    _wait_idx(0, 0)
    _issue_rows(0, 0)

    def _body(k2, _):
        i0 = 2 * k2
        i1 = i0 + 1
        _wait_idx(i1, 1)
        _issue_rows(i1, 1)
        _wait_in(i0, 0)
        _compute(i0, 0)
        _issue(i0 + 2, 0)
        _wait_idx(i0 + 2, 0)
        _issue_rows(i0 + 2, 0)
        _wait_in(i1, 1)
        _compute(i1, 1)
        _issue(i1 + 2, 1)
        return 0

    lax.fori_loop(0, (NCHUNK - 3) // 2, _body, 0)

    _wait_idx(NCHUNK - 2, 1)
    _issue_rows(NCHUNK - 2, 1)
    _wait_in(NCHUNK - 3, 0)
    _compute(NCHUNK - 3, 0)
    _issue(NCHUNK - 1, 0)
    _wait_idx(NCHUNK - 1, 0)
    _issue_rows(NCHUNK - 1, 0)
    _wait_in(NCHUNK - 2, 1)
    _compute(NCHUNK - 2, 1)
    _wait_in(NCHUNK - 1, 0)
    _compute(NCHUNK - 1, 0)

    plsc.subcore_barrier()

    @pl.when(s < NS - 1)
    def _():
        pltpu.sync_copy(acc_sh.at[pl.ds(s * STRIPE, STRIPE)],
                        acc_out.at[c, pl.ds(s * STRIPE, STRIPE)])
        pltpu.sync_copy(den_sh.at[pl.ds(s * STRIPE, STRIPE)], den_w)
        pltpu.sync_copy(den_w, den_out.at[pl.ds(c * N + s * STRIPE, STRIPE)])

    @pl.when(s == NS - 1)
    def _():
        pltpu.sync_copy(acc_sh.at[pl.ds(s * STRIPE, LAST)],
                        acc_out.at[c, pl.ds(s * STRIPE, LAST)])
        pltpu.sync_copy(den_sh.at[pl.ds(s * STRIPE, LAST)],
                        den_w.at[pl.ds(0, LAST)])
        pltpu.sync_copy(den_w.at[pl.ds(0, LAST)],
                        den_out.at[pl.ds(c * N + s * STRIPE, LAST)])


def kernel(x, edge_index, edge_attr, W_src, b_src, W_dst, b_dst,
           W_edge, b_edge, W_attn, b_attn, gamma, beta, alpha):
    src = edge_index[0]
    dst = edge_index[1]
    ea_flat = edge_attr.reshape(E * EA)
    w1 = W_attn[0:D]
    w2 = W_attn[D:2 * D]
    w3 = W_attn[2 * D:3 * D]

    hx, as_n, ad_n, we16 = pl.pallas_call(
        _node_proj_body,
        out_shape=[
            jax.ShapeDtypeStruct((N, D), jnp.float32),
            jax.ShapeDtypeStruct((N, 1), jnp.float32),
            jax.ShapeDtypeStruct((N, 1), jnp.float32),
            jax.ShapeDtypeStruct((16, 1), jnp.float32),
        ],
    )(x, W_src, b_src.reshape(1, D), w1, W_dst, w2, b_dst.reshape(1, D),
      W_edge, w3, b_edge.reshape(1, D), b_attn.reshape(1, 1))

    acc, den = _sc_aggregate(src, dst, ea_flat, as_n.reshape(N),
                             ad_n.reshape(N), hx, we16.reshape(16))

    out = pl.pallas_call(
        _finalize_body,
        out_shape=jax.ShapeDtypeStruct((N, D), jnp.float32),
    )(acc, den.reshape(NC, N, 1), gamma.reshape(1, D), beta.reshape(1, D),
      alpha.reshape(1, 1))
    return out
